# baseline (speedup 1.0000x reference)
import functools

import jax
import jax.numpy as jnp
from jax import lax
from jax.experimental import pallas as pl
from jax.experimental.pallas import tpu as pltpu
from jax.experimental.pallas import tpu_sc as plsc

_ROWS = 64
_NCOLS = 1_000_000

_SC_START = 770_048
_SC_COLS = _NCOLS - _SC_START
_SC_CHUNK = 57_488
_SC_NCHUNK = _SC_COLS // _SC_CHUNK
_SC_GROUPS = _SC_CHUNK // 16

_W = 8192
_SUB = 256
_NSUB = _W // _SUB
_GRID = (_SC_START + _W - 1) // _W

_KS1 = 1234
_KS2 = 1234 ^ 0x1BD11BDA
_M32 = 0xFFFFFFFF
_INJ = (
    (_KS1, (_KS2 + 1) & _M32),
    (_KS2, 2),
    (0, _KS1 + 3),
    (_KS1, (_KS2 + 4) & _M32),
    (_KS2, 5),
)
_ROT = ((13, 15, 26, 6), (17, 29, 16, 24))

_LN2 = 0.6931471805599453
_Q = (-0.0018304482800886035, 0.012968823313713074, -0.043113864958286285,
      0.09163002669811249, -0.1453178972005844, 0.19320762157440186,
      -0.2371523529291153, 0.2879810631275177, -0.360615611076355,
      0.4808950424194336, -0.721347451210022, 1.4426950216293335)


def _rotl(x, d):
    return (x << d) | (x >> (32 - d))


def _threefry_bits(a):
    x0 = a
    x1 = _rotl(a, 13) ^ a
    for d in (15, 26, 6):
        x0 = x0 + x1
        x1 = _rotl(x1, d) ^ x0
    x0 = x0 + jnp.uint32(_INJ[0][0])
    x1 = x1 + jnp.uint32(_INJ[0][1])
    for g in (1, 2, 3, 4):
        for d in _ROT[g % 2]:
            x0 = x0 + x1
            x1 = _rotl(x1, d) ^ x0
        if _INJ[g][0]:
            x0 = x0 + jnp.uint32(_INJ[g][0])
        x1 = x1 + jnp.uint32(_INJ[g][1])
    return x0 ^ x1


def _gumbel_val(bits, s):
    fb = (bits >> 9) | jnp.uint32(0x3F800000)
    u = jax.lax.bitcast_convert_type(fb, jnp.float32) - 1.0
    noise = jnp.maximum(-jnp.log1p(-u), 1e-10)
    return s - jnp.log(noise)


def _tc_body(logits_ref, temp_ref, idx_ref, val_ref, vmax_ref, va_ref):
    j = pl.program_id(0)
    rtemp = 1.0 / temp_ref[...]

    lane = jax.lax.broadcasted_iota(jnp.int32, (_ROWS, _SUB), 1)
    rowoff = jax.lax.broadcasted_iota(jnp.int32, (_ROWS, _SUB), 0) * _NCOLS
    a0 = (rowoff + lane + j * _W + _KS1).astype(jnp.uint32)

    vmax0 = jnp.where(j == 0, jnp.full((_ROWS, _SUB), -jnp.inf, jnp.float32),
                      vmax_ref[...])
    va0 = jnp.where(j == 0, jnp.zeros((_ROWS, _SUB), jnp.uint32),
                    va_ref[...])

    def sub(k, carry, masked):
        vmax, va, a = carry
        bits = _threefry_bits(a)
        s = logits_ref[:, pl.ds(k * _SUB, _SUB)] * rtemp
        val = _gumbel_val(bits, s)
        if masked:
            val = jnp.where(a < bound, val, -jnp.inf)
        upd = val > vmax
        return (jnp.where(upd, val, vmax), jnp.where(upd, a, va),
                a + jnp.uint32(_SUB))

    @pl.when(j < _GRID - 1)
    def _():
        vmax1, va1, _ = jax.lax.fori_loop(
            0, _NSUB, lambda k, c: sub(k, c, False), (vmax0, va0, a0),
            unroll=2)
        vmax_ref[...] = vmax1
        va_ref[...] = va1

    bound = (rowoff + (_SC_START + _KS1)).astype(jnp.uint32)

    @pl.when(j == _GRID - 1)
    def _():
        vmax1, va1, _ = jax.lax.fori_loop(
            0, _NSUB, lambda k, c: sub(k, c, True), (vmax0, va0, a0),
            unroll=2)
        rmax = jnp.max(vmax1, axis=1, keepdims=True)
        col = (va1.astype(jnp.int32) - _KS1) - rowoff
        cand = jnp.where(vmax1 == rmax, col, jnp.int32(2**31 - 1))
        idx_ref[...] = jnp.min(cand, axis=1, keepdims=True)
        val_ref[...] = rmax


def _sc_log2(x):
    xb = jax.lax.bitcast_convert_type(x, jnp.int32)
    e = (xb >> 23) - 127
    t = jax.lax.bitcast_convert_type(
        (xb & 0x7FFFFF) | 0x3F800000, jnp.float32) - 1.0
    acc = jnp.float32(_Q[0])
    for c in _Q[1:]:
        acc = acc * t + jnp.float32(c)
    return e.astype(jnp.float32) + t * acc


def _sc_val(bits, s):
    fb = (bits >> jnp.uint32(9)) | jnp.uint32(0x3F800000)
    f = jax.lax.bitcast_convert_type(fb, jnp.float32)
    u = f - 1.0
    w = 2.0 - f
    acc = jnp.float32(1.0 / 8.0)
    for k in (7, 6, 5, 4, 3, 2, 1):
        acc = acc * u + jnp.float32(1.0 / k)
    noise_s = u * acc
    noise_f = jnp.float32(-_LN2) * _sc_log2(jnp.maximum(w, 1e-30))
    noise = jnp.where(u < 0.125, noise_s, noise_f)
    noise = jnp.maximum(noise, 1e-10)
    return s - jnp.float32(_LN2) * _sc_log2(noise)


def _sc_kernel_body(tail_hbm, invtemp_hbm, val_out, col_out,
                    buf0, buf1, tbuf, stage_f, stage_i):
    wid = lax.axis_index("s") * 2 + lax.axis_index("c")
    r0 = wid * 2
    pltpu.sync_copy(invtemp_hbm, tbuf.at[pl.ds(0, _ROWS)])
    tv = tbuf[pl.ds(r0, 16)]
    inv0 = tv[0]
    inv1 = tv[1]

    lane16 = jax.lax.iota(jnp.uint32, 16)
    a_init = lane16 + jnp.uint32(_KS1 + _SC_START) + (
        jnp.uint32(r0) * jnp.uint32(_NCOLS))
    neg = jnp.full((16,), -jnp.inf, jnp.float32)
    zero = jnp.zeros((16,), jnp.uint32)
    carry = (neg, zero, neg, zero, a_init)

    def group(g, c):
        vm0, va0, vm1, va1, a = c
        s0 = buf0[pl.ds(g * 16, 16)] * inv0
        s1 = buf1[pl.ds(g * 16, 16)] * inv1
        v0 = _sc_val(_threefry_bits(a), s0)
        a1 = a + jnp.uint32(_NCOLS)
        v1 = _sc_val(_threefry_bits(a1), s1)
        up0 = v0 > vm0
        up1 = v1 > vm1
        return (jnp.where(up0, v0, vm0), jnp.where(up0, a, va0),
                jnp.where(up1, v1, vm1), jnp.where(up1, a1, va1),
                a + jnp.uint32(16))

    for ch in range(_SC_NCHUNK):
        c0 = ch * _SC_CHUNK
        pltpu.sync_copy(tail_hbm.at[pl.ds(r0 * _SC_COLS + c0, _SC_CHUNK)],
                        buf0)
        pltpu.sync_copy(tail_hbm.at[pl.ds((r0 + 1) * _SC_COLS + c0,
                                          _SC_CHUNK)], buf1)
        carry = jax.lax.fori_loop(0, _SC_GROUPS, group, carry, unroll=2)

    vm0, va0, vm1, va1, _ = carry
    base0 = jnp.uint32(r0) * jnp.uint32(_NCOLS) + jnp.uint32(_KS1)
    stage_f[...] = vm0
    pltpu.sync_copy(stage_f, val_out.at[pl.ds(r0 * 16, 16)])
    stage_i[...] = (va0 - base0).astype(jnp.int32)
    pltpu.sync_copy(stage_i, col_out.at[pl.ds(r0 * 16, 16)])
    stage_f[...] = vm1
    pltpu.sync_copy(stage_f, val_out.at[pl.ds((r0 + 1) * 16, 16)])
    stage_i[...] = (va1 - base0 - jnp.uint32(_NCOLS)).astype(jnp.int32)
    pltpu.sync_copy(stage_i, col_out.at[pl.ds((r0 + 1) * 16, 16)])


@functools.cache
def _sc_sample_fn():
    return functools.partial(
        pl.kernel,
        out_type=[jax.ShapeDtypeStruct((_ROWS * 16,), jnp.float32),
                  jax.ShapeDtypeStruct((_ROWS * 16,), jnp.int32)],
        mesh=plsc.VectorSubcoreMesh(core_axis_name="c", subcore_axis_name="s"),
        scratch_types=[pltpu.VMEM((_SC_CHUNK,), jnp.float32),
                       pltpu.VMEM((_SC_CHUNK,), jnp.float32),
                       pltpu.VMEM((_ROWS + 16,), jnp.float32),
                       pltpu.VMEM((16,), jnp.float32),
                       pltpu.VMEM((16,), jnp.int32)],
    )(_sc_kernel_body)


def _merge_body(tcv_ref, tci_ref, scv_ref, sci_ref, out_ref):
    v = jnp.concatenate([tcv_ref[...], scv_ref[...]], axis=1)
    c = jnp.concatenate([tci_ref[...], sci_ref[...]], axis=1)
    m = jnp.max(v, axis=1, keepdims=True)
    cand = jnp.where(v == m, c, jnp.int32(2**31 - 1))
    out_ref[...] = jnp.min(cand, axis=1, keepdims=True)


@functools.partial(jax.jit, static_argnames=("interpret",))
def _sample(logits, temperature, interpret=False):
    tci, tcv = pl.pallas_call(
        _tc_body,
        grid=(_GRID,),
        in_specs=[
            pl.BlockSpec((_ROWS, _W), lambda j: (0, j)),
            pl.BlockSpec((_ROWS, 1), lambda j: (0, 0)),
        ],
        out_specs=[pl.BlockSpec((_ROWS, 1), lambda j: (0, 0)),
                   pl.BlockSpec((_ROWS, 1), lambda j: (0, 0))],
        out_shape=[jax.ShapeDtypeStruct((_ROWS, 1), jnp.int32),
                   jax.ShapeDtypeStruct((_ROWS, 1), jnp.float32)],
        scratch_shapes=[
            pltpu.VMEM((_ROWS, _SUB), jnp.float32),
            pltpu.VMEM((_ROWS, _SUB), jnp.uint32),
        ],
        interpret=interpret,
    )(logits, temperature.reshape(_ROWS, 1))

    tail = logits[:, _SC_START:].reshape(_ROWS * _SC_COLS)
    scv, sci = _sc_sample_fn()(tail, 1.0 / temperature)
    scv = scv.reshape(_ROWS, 16)
    sci = sci.reshape(_ROWS, 16)

    idx = pl.pallas_call(
        _merge_body,
        out_shape=jax.ShapeDtypeStruct((_ROWS, 1), jnp.int32),
        interpret=interpret,
    )(tcv, tci, scv, sci)
    return idx[:, 0]


def kernel(logits, temperature):
    return _sample(logits, temperature)


# speedup vs baseline: 1.2421x; 1.2421x over previous
import functools

import jax
import jax.numpy as jnp
from jax import lax
from jax.experimental import pallas as pl
from jax.experimental.pallas import tpu as pltpu
from jax.experimental.pallas import tpu_sc as plsc

_ROWS = 64
_NCOLS = 1_000_000

_SC_START = 823_872
_SC_COLS = _NCOLS - _SC_START
_SC_CHUNK = 44_032
_SC_NCHUNK = _SC_COLS // _SC_CHUNK
_SC_GROUPS = _SC_CHUNK // 16

_W = 8192
_SUB = 256
_NSUB = _W // _SUB
_GRID = (_SC_START + _W - 1) // _W

_KS1 = 1234
_KS2 = 1234 ^ 0x1BD11BDA
_M32 = 0xFFFFFFFF
_INJ = (
    (_KS1, (_KS2 + 1) & _M32),
    (_KS2, 2),
    (0, _KS1 + 3),
    (_KS1, (_KS2 + 4) & _M32),
    (_KS2, 5),
)
_ROT = ((13, 15, 26, 6), (17, 29, 16, 24))

_LN2 = 0.6931471805599453
_Q = (-0.0018304482800886035, 0.012968823313713074, -0.043113864958286285,
      0.09163002669811249, -0.1453178972005844, 0.19320762157440186,
      -0.2371523529291153, 0.2879810631275177, -0.360615611076355,
      0.4808950424194336, -0.721347451210022, 1.4426950216293335)


def _rotl(x, d):
    return (x << d) | (x >> (32 - d))


def _threefry_bits(a):
    x0 = a
    x1 = _rotl(a, 13) ^ a
    for d in (15, 26, 6):
        x0 = x0 + x1
        x1 = _rotl(x1, d) ^ x0
    x0 = x0 + jnp.uint32(_INJ[0][0])
    x1 = x1 + jnp.uint32(_INJ[0][1])
    for g in (1, 2, 3, 4):
        for d in _ROT[g % 2]:
            x0 = x0 + x1
            x1 = _rotl(x1, d) ^ x0
        if _INJ[g][0]:
            x0 = x0 + jnp.uint32(_INJ[g][0])
        x1 = x1 + jnp.uint32(_INJ[g][1])
    return x0 ^ x1


def _gumbel_val(bits, s):
    fb = (bits >> 9) | jnp.uint32(0x3F800000)
    u = jax.lax.bitcast_convert_type(fb, jnp.float32) - 1.0
    noise = jnp.maximum(-jnp.log1p(-u), 1e-10)
    return s - jnp.log(noise)


def _tc_body(logits_ref, temp_ref, idx_ref, val_ref, vmax_ref, va_ref):
    j = pl.program_id(0)
    rtemp = 1.0 / temp_ref[...]

    lane = jax.lax.broadcasted_iota(jnp.int32, (_ROWS, _SUB), 1)
    rowoff = jax.lax.broadcasted_iota(jnp.int32, (_ROWS, _SUB), 0) * _NCOLS
    a0 = (rowoff + lane + j * _W + _KS1).astype(jnp.uint32)

    vmax0 = jnp.where(j == 0, jnp.full((_ROWS, _SUB), -jnp.inf, jnp.float32),
                      vmax_ref[...])
    va0 = jnp.where(j == 0, jnp.zeros((_ROWS, _SUB), jnp.uint32),
                    va_ref[...])

    def sub(k, carry, masked):
        vmax, va, a = carry
        bits = _threefry_bits(a)
        s = logits_ref[:, pl.ds(k * _SUB, _SUB)] * rtemp
        val = _gumbel_val(bits, s)
        if masked:
            val = jnp.where(a < bound, val, -jnp.inf)
        upd = val > vmax
        return (jnp.where(upd, val, vmax), jnp.where(upd, a, va),
                a + jnp.uint32(_SUB))

    @pl.when(j < _GRID - 1)
    def _():
        vmax1, va1, _ = jax.lax.fori_loop(
            0, _NSUB, lambda k, c: sub(k, c, False), (vmax0, va0, a0),
            unroll=2)
        vmax_ref[...] = vmax1
        va_ref[...] = va1

    bound = (rowoff + (_SC_START + _KS1)).astype(jnp.uint32)

    @pl.when(j == _GRID - 1)
    def _():
        vmax1, va1, _ = jax.lax.fori_loop(
            0, _NSUB, lambda k, c: sub(k, c, True), (vmax0, va0, a0),
            unroll=2)
        rmax = jnp.max(vmax1, axis=1, keepdims=True)
        col = (va1.astype(jnp.int32) - _KS1) - rowoff
        cand = jnp.where(vmax1 == rmax, col, jnp.int32(2**31 - 1))
        idx_ref[...] = jnp.min(cand, axis=1, keepdims=True)
        val_ref[...] = rmax


def _sc_log2(x):
    xb = jax.lax.bitcast_convert_type(x, jnp.int32)
    e = (xb >> 23) - 127
    t = jax.lax.bitcast_convert_type(
        (xb & 0x7FFFFF) | 0x3F800000, jnp.float32) - 1.0
    acc = jnp.float32(_Q[0])
    for c in _Q[1:]:
        acc = acc * t + jnp.float32(c)
    return e.astype(jnp.float32) + t * acc


def _sc_val(bits, s):
    fb = (bits >> jnp.uint32(9)) | jnp.uint32(0x3F800000)
    f = jax.lax.bitcast_convert_type(fb, jnp.float32)
    u = f - 1.0
    w = 2.0 - f
    acc = jnp.float32(1.0 / 8.0)
    for k in (7, 6, 5, 4, 3, 2, 1):
        acc = acc * u + jnp.float32(1.0 / k)
    noise_s = u * acc
    noise_f = jnp.float32(-_LN2) * _sc_log2(jnp.maximum(w, 1e-30))
    noise = jnp.where(u < 0.125, noise_s, noise_f)
    noise = jnp.maximum(noise, 1e-10)
    return s - jnp.float32(_LN2) * _sc_log2(noise)


def _sc_kernel_body(tail_hbm, invtemp_hbm, val_out, col_out,
                    buf0, buf1, tbuf, stage_f, stage_i):
    wid = lax.axis_index("s") * 2 + lax.axis_index("c")
    r0 = wid * 2
    pltpu.sync_copy(invtemp_hbm, tbuf.at[pl.ds(0, _ROWS)])
    tv = tbuf[pl.ds(r0, 16)]
    inv0 = tv[0]
    inv1 = tv[1]

    lane16 = jax.lax.iota(jnp.uint32, 16)
    a_init = lane16 + jnp.uint32(_KS1 + _SC_START) + (
        jnp.uint32(r0) * jnp.uint32(_NCOLS))
    neg = jnp.full((16,), -jnp.inf, jnp.float32)
    zero = jnp.zeros((16,), jnp.uint32)
    carry = (neg, zero, neg, zero, a_init)

    def group(g, c):
        vm0, va0, vm1, va1, a = c
        s0 = buf0[pl.ds(g * 16, 16)] * inv0
        s1 = buf1[pl.ds(g * 16, 16)] * inv1
        v0 = _sc_val(_threefry_bits(a), s0)
        a1 = a + jnp.uint32(_NCOLS)
        v1 = _sc_val(_threefry_bits(a1), s1)
        up0 = v0 > vm0
        up1 = v1 > vm1
        return (jnp.where(up0, v0, vm0), jnp.where(up0, a, va0),
                jnp.where(up1, v1, vm1), jnp.where(up1, a1, va1),
                a + jnp.uint32(16))

    for ch in range(_SC_NCHUNK):
        c0 = ch * _SC_CHUNK
        pltpu.sync_copy(tail_hbm.at[pl.ds(r0 * _SC_COLS + c0, _SC_CHUNK)],
                        buf0)
        pltpu.sync_copy(tail_hbm.at[pl.ds((r0 + 1) * _SC_COLS + c0,
                                          _SC_CHUNK)], buf1)
        carry = jax.lax.fori_loop(0, _SC_GROUPS, group, carry, unroll=2)

    vm0, va0, vm1, va1, _ = carry
    base0 = jnp.uint32(r0) * jnp.uint32(_NCOLS) + jnp.uint32(_KS1)
    stage_f[...] = vm0
    pltpu.sync_copy(stage_f, val_out.at[pl.ds(r0 * 16, 16)])
    stage_i[...] = (va0 - base0).astype(jnp.int32)
    pltpu.sync_copy(stage_i, col_out.at[pl.ds(r0 * 16, 16)])
    stage_f[...] = vm1
    pltpu.sync_copy(stage_f, val_out.at[pl.ds((r0 + 1) * 16, 16)])
    stage_i[...] = (va1 - base0 - jnp.uint32(_NCOLS)).astype(jnp.int32)
    pltpu.sync_copy(stage_i, col_out.at[pl.ds((r0 + 1) * 16, 16)])


@functools.cache
def _sc_sample_fn():
    return functools.partial(
        pl.kernel,
        out_type=[jax.ShapeDtypeStruct((_ROWS * 16,), jnp.float32),
                  jax.ShapeDtypeStruct((_ROWS * 16,), jnp.int32)],
        mesh=plsc.VectorSubcoreMesh(core_axis_name="c", subcore_axis_name="s"),
        scratch_types=[pltpu.VMEM((_SC_CHUNK,), jnp.float32),
                       pltpu.VMEM((_SC_CHUNK,), jnp.float32),
                       pltpu.VMEM((_ROWS + 16,), jnp.float32),
                       pltpu.VMEM((16,), jnp.float32),
                       pltpu.VMEM((16,), jnp.int32)],
    )(_sc_kernel_body)


def _merge_body(tcv_ref, tci_ref, scv_ref, sci_ref, out_ref):
    v = jnp.concatenate([tcv_ref[...], scv_ref[...]], axis=1)
    c = jnp.concatenate([tci_ref[...], sci_ref[...]], axis=1)
    m = jnp.max(v, axis=1, keepdims=True)
    cand = jnp.where(v == m, c, jnp.int32(2**31 - 1))
    out_ref[...] = jnp.min(cand, axis=1, keepdims=True)


@functools.partial(jax.jit, static_argnames=("interpret",))
def _sample(logits, temperature, interpret=False):
    tci, tcv = pl.pallas_call(
        _tc_body,
        grid=(_GRID,),
        in_specs=[
            pl.BlockSpec((_ROWS, _W), lambda j: (0, j)),
            pl.BlockSpec((_ROWS, 1), lambda j: (0, 0)),
        ],
        out_specs=[pl.BlockSpec((_ROWS, 1), lambda j: (0, 0)),
                   pl.BlockSpec((_ROWS, 1), lambda j: (0, 0))],
        out_shape=[jax.ShapeDtypeStruct((_ROWS, 1), jnp.int32),
                   jax.ShapeDtypeStruct((_ROWS, 1), jnp.float32)],
        scratch_shapes=[
            pltpu.VMEM((_ROWS, _SUB), jnp.float32),
            pltpu.VMEM((_ROWS, _SUB), jnp.uint32),
        ],
        interpret=interpret,
    )(logits, temperature.reshape(_ROWS, 1))

    tail = logits[:, _SC_START:].reshape(_ROWS * _SC_COLS)
    scv, sci = _sc_sample_fn()(tail, 1.0 / temperature)
    scv = scv.reshape(_ROWS, 16)
    sci = sci.reshape(_ROWS, 16)

    idx = pl.pallas_call(
        _merge_body,
        out_shape=jax.ShapeDtypeStruct((_ROWS, 1), jnp.int32),
        interpret=interpret,
    )(tcv, tci, scv, sci)
    return idx[:, 0]


def kernel(logits, temperature):
    return _sample(logits, temperature)


# speedup vs baseline: 1.2427x; 1.0005x over previous
import functools

import jax
import jax.numpy as jnp
from jax import lax
from jax.experimental import pallas as pl
from jax.experimental.pallas import tpu as pltpu
from jax.experimental.pallas import tpu_sc as plsc

_ROWS = 64
_NCOLS = 1_000_000

_SC_START = 823_872
_SC_COLS = _NCOLS - _SC_START
_SC_CHUNK = 44_032
_SC_NCHUNK = _SC_COLS // _SC_CHUNK
_SC_GROUPS = _SC_CHUNK // 16

_W = 8192
_SUB = 256
_NSUB = _W // _SUB
_GRID = (_SC_START + _W - 1) // _W

_KS1 = 1234
_KS2 = 1234 ^ 0x1BD11BDA
_M32 = 0xFFFFFFFF
_INJ = (
    (_KS1, (_KS2 + 1) & _M32),
    (_KS2, 2),
    (0, _KS1 + 3),
    (_KS1, (_KS2 + 4) & _M32),
    (_KS2, 5),
)
_ROT = ((13, 15, 26, 6), (17, 29, 16, 24))

_LN2 = 0.6931471805599453
_Q = (-0.0018304482800886035, 0.012968823313713074, -0.043113864958286285,
      0.09163002669811249, -0.1453178972005844, 0.19320762157440186,
      -0.2371523529291153, 0.2879810631275177, -0.360615611076355,
      0.4808950424194336, -0.721347451210022, 1.4426950216293335)


def _rotl(x, d):
    return (x << d) | (x >> (32 - d))


def _threefry_bits(a):
    x0 = a
    x1 = _rotl(a, 13) ^ a
    for d in (15, 26, 6):
        x0 = x0 + x1
        x1 = _rotl(x1, d) ^ x0
    x0 = x0 + jnp.uint32(_INJ[0][0])
    x1 = x1 + jnp.uint32(_INJ[0][1])
    for g in (1, 2, 3, 4):
        for d in _ROT[g % 2]:
            x0 = x0 + x1
            x1 = _rotl(x1, d) ^ x0
        if _INJ[g][0]:
            x0 = x0 + jnp.uint32(_INJ[g][0])
        x1 = x1 + jnp.uint32(_INJ[g][1])
    return x0 ^ x1


def _gumbel_val(bits, s):
    fb = (bits >> 9) | jnp.uint32(0x3F800000)
    u = jax.lax.bitcast_convert_type(fb, jnp.float32) - 1.0
    noise = jnp.maximum(-jnp.log1p(-u), 1e-10)
    return s - jnp.log(noise)


def _tc_body(logits_ref, temp_ref, idx_ref, val_ref, vmax_ref, va_ref):
    j = pl.program_id(0)
    rtemp = 1.0 / temp_ref[...]

    lane = jax.lax.broadcasted_iota(jnp.int32, (_ROWS, _SUB), 1)
    rowoff = jax.lax.broadcasted_iota(jnp.int32, (_ROWS, _SUB), 0) * _NCOLS
    a0 = (rowoff + lane + j * _W + _KS1).astype(jnp.uint32)

    vmax0 = jnp.where(j == 0, jnp.full((_ROWS, _SUB), -jnp.inf, jnp.float32),
                      vmax_ref[...])
    va0 = jnp.where(j == 0, jnp.zeros((_ROWS, _SUB), jnp.uint32),
                    va_ref[...])

    def sub(k, carry, masked):
        vmax, va, a = carry
        bits = _threefry_bits(a)
        s = logits_ref[:, pl.ds(k * _SUB, _SUB)] * rtemp
        val = _gumbel_val(bits, s)
        if masked:
            val = jnp.where(a < bound, val, -jnp.inf)
        upd = val > vmax
        return (jnp.where(upd, val, vmax), jnp.where(upd, a, va),
                a + jnp.uint32(_SUB))

    @pl.when(j < _GRID - 1)
    def _():
        vmax1, va1, _ = jax.lax.fori_loop(
            0, _NSUB, lambda k, c: sub(k, c, False), (vmax0, va0, a0),
            unroll=2)
        vmax_ref[...] = vmax1
        va_ref[...] = va1

    bound = (rowoff + (_SC_START + _KS1)).astype(jnp.uint32)

    @pl.when(j == _GRID - 1)
    def _():
        vmax1, va1, _ = jax.lax.fori_loop(
            0, _NSUB, lambda k, c: sub(k, c, True), (vmax0, va0, a0),
            unroll=2)
        rmax = jnp.max(vmax1, axis=1, keepdims=True)
        col = (va1.astype(jnp.int32) - _KS1) - rowoff
        cand = jnp.where(vmax1 == rmax, col, jnp.int32(2**31 - 1))
        idx_ref[...] = jnp.min(cand, axis=1, keepdims=True)
        val_ref[...] = rmax


def _sc_log2(x):
    xb = jax.lax.bitcast_convert_type(x, jnp.int32)
    e = (xb >> 23) - 127
    t = jax.lax.bitcast_convert_type(
        (xb & 0x7FFFFF) | 0x3F800000, jnp.float32) - 1.0
    acc = jnp.float32(_Q[0])
    for c in _Q[1:]:
        acc = acc * t + jnp.float32(c)
    return e.astype(jnp.float32) + t * acc


def _sc_val(bits, s):
    fb = (bits >> jnp.uint32(9)) | jnp.uint32(0x3F800000)
    f = jax.lax.bitcast_convert_type(fb, jnp.float32)
    u = f - 1.0
    w = 2.0 - f
    acc = jnp.float32(1.0 / 8.0)
    for k in (7, 6, 5, 4, 3, 2, 1):
        acc = acc * u + jnp.float32(1.0 / k)
    noise_s = u * acc
    noise_f = jnp.float32(-_LN2) * _sc_log2(jnp.maximum(w, 1e-30))
    noise = jnp.where(u < 0.125, noise_s, noise_f)
    noise = jnp.maximum(noise, 1e-10)
    return s - jnp.float32(_LN2) * _sc_log2(noise)


def _sc_kernel_body(tail_hbm, invtemp_hbm, val_out, col_out,
                    buf0, buf1, tbuf, stage_f, stage_i):
    wid = lax.axis_index("s") * 2 + lax.axis_index("c")
    r0 = wid * 2
    pltpu.sync_copy(invtemp_hbm, tbuf.at[pl.ds(0, _ROWS)])
    tv = tbuf[pl.ds(r0, 16)]
    inv0 = tv[0]
    inv1 = tv[1]

    lane16 = jax.lax.iota(jnp.uint32, 16)
    a_init = lane16 + jnp.uint32(_KS1 + _SC_START) + (
        jnp.uint32(r0) * jnp.uint32(_NCOLS))
    neg = jnp.full((16,), -jnp.inf, jnp.float32)
    zero = jnp.zeros((16,), jnp.uint32)
    carry = (neg, zero, neg, zero, a_init)

    def group(g, c):
        vm0, va0, vm1, va1, a = c
        s0 = buf0[pl.ds(g * 16, 16)] * inv0
        s1 = buf1[pl.ds(g * 16, 16)] * inv1
        v0 = _sc_val(_threefry_bits(a), s0)
        a1 = a + jnp.uint32(_NCOLS)
        v1 = _sc_val(_threefry_bits(a1), s1)
        up0 = v0 > vm0
        up1 = v1 > vm1
        return (jnp.where(up0, v0, vm0), jnp.where(up0, a, va0),
                jnp.where(up1, v1, vm1), jnp.where(up1, a1, va1),
                a + jnp.uint32(16))

    for ch in range(_SC_NCHUNK):
        c0 = ch * _SC_CHUNK
        pltpu.sync_copy(tail_hbm.at[pl.ds(r0 * _SC_COLS + c0, _SC_CHUNK)],
                        buf0)
        pltpu.sync_copy(tail_hbm.at[pl.ds((r0 + 1) * _SC_COLS + c0,
                                          _SC_CHUNK)], buf1)
        carry = jax.lax.fori_loop(0, _SC_GROUPS, group, carry, unroll=4)

    vm0, va0, vm1, va1, _ = carry
    base0 = jnp.uint32(r0) * jnp.uint32(_NCOLS) + jnp.uint32(_KS1)
    stage_f[...] = vm0
    pltpu.sync_copy(stage_f, val_out.at[pl.ds(r0 * 16, 16)])
    stage_i[...] = (va0 - base0).astype(jnp.int32)
    pltpu.sync_copy(stage_i, col_out.at[pl.ds(r0 * 16, 16)])
    stage_f[...] = vm1
    pltpu.sync_copy(stage_f, val_out.at[pl.ds((r0 + 1) * 16, 16)])
    stage_i[...] = (va1 - base0 - jnp.uint32(_NCOLS)).astype(jnp.int32)
    pltpu.sync_copy(stage_i, col_out.at[pl.ds((r0 + 1) * 16, 16)])


@functools.cache
def _sc_sample_fn():
    return functools.partial(
        pl.kernel,
        out_type=[jax.ShapeDtypeStruct((_ROWS * 16,), jnp.float32),
                  jax.ShapeDtypeStruct((_ROWS * 16,), jnp.int32)],
        mesh=plsc.VectorSubcoreMesh(core_axis_name="c", subcore_axis_name="s"),
        scratch_types=[pltpu.VMEM((_SC_CHUNK,), jnp.float32),
                       pltpu.VMEM((_SC_CHUNK,), jnp.float32),
                       pltpu.VMEM((_ROWS + 16,), jnp.float32),
                       pltpu.VMEM((16,), jnp.float32),
                       pltpu.VMEM((16,), jnp.int32)],
    )(_sc_kernel_body)


def _merge_body(tcv_ref, tci_ref, scv_ref, sci_ref, out_ref):
    v = jnp.concatenate([tcv_ref[...], scv_ref[...]], axis=1)
    c = jnp.concatenate([tci_ref[...], sci_ref[...]], axis=1)
    m = jnp.max(v, axis=1, keepdims=True)
    cand = jnp.where(v == m, c, jnp.int32(2**31 - 1))
    out_ref[...] = jnp.min(cand, axis=1, keepdims=True)


@functools.partial(jax.jit, static_argnames=("interpret",))
def _sample(logits, temperature, interpret=False):
    tci, tcv = pl.pallas_call(
        _tc_body,
        grid=(_GRID,),
        in_specs=[
            pl.BlockSpec((_ROWS, _W), lambda j: (0, j)),
            pl.BlockSpec((_ROWS, 1), lambda j: (0, 0)),
        ],
        out_specs=[pl.BlockSpec((_ROWS, 1), lambda j: (0, 0)),
                   pl.BlockSpec((_ROWS, 1), lambda j: (0, 0))],
        out_shape=[jax.ShapeDtypeStruct((_ROWS, 1), jnp.int32),
                   jax.ShapeDtypeStruct((_ROWS, 1), jnp.float32)],
        scratch_shapes=[
            pltpu.VMEM((_ROWS, _SUB), jnp.float32),
            pltpu.VMEM((_ROWS, _SUB), jnp.uint32),
        ],
        interpret=interpret,
    )(logits, temperature.reshape(_ROWS, 1))

    tail = logits[:, _SC_START:].reshape(_ROWS * _SC_COLS)
    scv, sci = _sc_sample_fn()(tail, 1.0 / temperature)
    scv = scv.reshape(_ROWS, 16)
    sci = sci.reshape(_ROWS, 16)

    idx = pl.pallas_call(
        _merge_body,
        out_shape=jax.ShapeDtypeStruct((_ROWS, 1), jnp.int32),
        interpret=interpret,
    )(tcv, tci, scv, sci)
    return idx[:, 0]


def kernel(logits, temperature):
    return _sample(logits, temperature)
